# Initial kernel scaffold; baseline (speedup 1.0000x reference)
import jax
import jax.numpy as jnp
from jax.experimental import pallas as pl


def kernel(x, edge_index, edge_attr, num_graphs, Wr1, br1, Wo1, Wr2, br2, Wo2, Wr3, br3, Wo3, bn1_w, bn1_b, bn2_w, bn2_b, fc1_W, fc1_b, fc2_W, fc2_b):
    raise NotImplementedError("write your pallas kernel here")


# speedup vs baseline: 9.4624x; 9.4624x over previous
import functools

import jax
import jax.numpy as jnp
from jax import lax
from jax.experimental import pallas as pl
from jax.experimental.pallas import tpu as pltpu
from jax.experimental.pallas import tpu_sc as plsc

N = 10000
E = 320000
G = 4
NPG = 2500
H = 20
SLOPE = 0.01
EPS_BN = 1e-5

EROWS = 2560
LANES = 16


def _tc_in_body(x_ref, wr_ref, wo_ref, br_ref, y_ref, o_ref):
    xv = x_ref[...]
    dn = (((0,), (1,)), ((), ()))
    y_ref[...] = lax.dot_general(wr_ref[...], xv, dn,
                                 preferred_element_type=jnp.float32)
    o_ref[...] = lax.dot_general(wo_ref[...], xv, dn,
                                 preferred_element_type=jnp.float32) \
        + br_ref[...][:, None]


def _tc_in(x, wr, wo, br):
    return pl.pallas_call(
        _tc_in_body,
        out_shape=[jax.ShapeDtypeStruct((H, N), jnp.float32),
                   jax.ShapeDtypeStruct((H, N), jnp.float32)],
    )(x, wr, wo, br)


def _tc_mid_body(kn, agg_ref, obr_ref, bnw_ref, bnb_ref, wr_ref, wo_ref,
                 brn_ref, y_ref, o_ref):
    h = agg_ref[...] + obr_ref[...]
    h = jnp.where(h >= 0, h, SLOPE * h)
    d0 = h[:, 0:NPG]
    d1 = h[:, NPG:2 * NPG]
    d2 = h[:, 2 * NPG:3 * NPG]
    d3 = h[:, 3 * NPG:4 * NPG]
    inv = 1.0 / (4.0 * H)
    mean = jnp.sum(d0 + d1 + d2 + d3, axis=0, keepdims=True) * inv
    e0 = d0 - mean
    e1 = d1 - mean
    e2 = d2 - mean
    e3 = d3 - mean
    var = jnp.sum(e0 * e0 + e1 * e1 + e2 * e2 + e3 * e3,
                  axis=0, keepdims=True) * inv
    scale = lax.rsqrt(var + EPS_BN) * bnw_ref[...][None, :]
    shift = bnb_ref[...][None, :]
    dn = (((0,), (0,)), ((), ()))
    es = (e0, e1, e2, e3)
    for g in range(G):
        hg = es[g] * scale + shift
        y_ref[:, g * NPG:(g + 1) * NPG] = lax.dot_general(
            wr_ref[...], hg, dn, preferred_element_type=jnp.float32)
        o_ref[:, g * NPG:(g + 1) * NPG] = lax.dot_general(
            wo_ref[...], hg, dn, preferred_element_type=jnp.float32) \
            + brn_ref[...][:, None]


def _tc_mid(agg, obr, bnw, bnb, wr_n, wo_n, br_n):
    kn = wr_n.shape[1]
    return pl.pallas_call(
        functools.partial(_tc_mid_body, kn),
        out_shape=[jax.ShapeDtypeStruct((kn, N), jnp.float32),
                   jax.ShapeDtypeStruct((kn, N), jnp.float32)],
    )(agg, obr, bnw, bnb, wr_n, wo_n, br_n)


def _tc_out_body(agg_ref, obr_ref, f1w_ref, f1b_ref, f2w_ref, f2b_ref,
                 out_ref, h_ref):
    hrow = agg_ref[0:1, :] + agg_ref[1:2, :] + obr_ref[...]
    hrow = jnp.where(hrow >= 0, hrow, SLOPE * hrow)
    for g in range(G):
        hg = hrow[:, g * NPG:(g + 1) * NPG]
        h_ref[g:g + 1, :] = hg
        m = jnp.sum(hg) * (1.0 / NPG)
        d = hg - m
        v = jnp.sum(d * d) * (1.0 / (NPG - 1))
        nrm = d / (v + 1e-10)
        dn = (((1,), (0,)), ((), ()))
        t = lax.dot_general(nrm, f1w_ref[...], dn,
                            preferred_element_type=jnp.float32) \
            + f1b_ref[...][None, :]
        t = jnp.where(t >= 0, t, SLOPE * t)
        out_ref[g:g + 1, :] = lax.dot_general(
            t, f2w_ref[...], dn, preferred_element_type=jnp.float32) \
            + f2b_ref[...][None, :]


def _tc_out(agg3, obr3, f1w, f1b, f2w, f2b):
    return pl.pallas_call(
        _tc_out_body,
        out_shape=[jax.ShapeDtypeStruct((G, 10), jnp.float32),
                   jax.ShapeDtypeStruct((G, NPG), jnp.float32)],
    )(agg3, obr3, f1w, f1b, f2w, f2b)


def _make_sc_agg(kc, cg, ngrp, nshard, chunk_rows):
    acc_rows = ngrp * cg
    rows_per_shard = EROWS // (2 * nshard) if kc == 1 else EROWS // nshard
    nchunk = rows_per_shard // chunk_rows
    assert rows_per_shard % chunk_rows == 0
    out_rows = kc if kc > 1 else 2

    mesh = plsc.VectorSubcoreMesh(core_axis_name="c", subcore_axis_name="s",
                                  num_cores=2, num_subcores=16)

    scratch = [
        pltpu.VMEM((cg * N,), jnp.float32),
        pltpu.VMEM((chunk_rows, 128), jnp.int32),
        pltpu.VMEM((chunk_rows, 1, 128), jnp.int32),
        pltpu.VMEM((chunk_rows, 128), jnp.float32),
        pltpu.VMEM((cg, chunk_rows, 1, 128), jnp.float32),
        pltpu.VMEM((N,), jnp.float32),
        pltpu.SemaphoreType.DMA,
        pltpu.SemaphoreType.DMA,
    ] + [pltpu.VMEM_SHARED((N,), jnp.float32) for _ in range(acc_rows)]

    @functools.partial(
        pl.kernel,
        out_type=jax.ShapeDtypeStruct((out_rows * N,), jnp.float32),
        mesh=mesh,
        scratch_types=scratch,
        compiler_params=pltpu.CompilerParams(needs_layout_passes=False),
    )
    def sc_agg(y_hbm, src_hbm, dst_hbm, ew_hbm, zero_hbm, out_hbm,
               zv, sbuf, dbuf, wbuf, msg, bounce, sem_e, sem_sc, *accs):
        c = lax.axis_index("c")
        s = lax.axis_index("s")

        @pl.when(s == 0)
        def _():
            pltpu.sync_copy(zero_hbm, bounce)
            for r in range(acc_rows):
                pltpu.sync_copy(bounce, accs[r])

        if kc == 1:
            grp = s - s
            shard = c * nshard + s
            col0 = 0
        else:
            grp = s // nshard
            shard = s % nshard
            col0 = c * acc_rows + grp * cg

        for cc in range(cg):
            off = pl.multiple_of((col0 + cc) * N, 8)
            pltpu.sync_copy(y_hbm.at[pl.ds(off, N)], zv.at[pl.ds(cc * N, N)])
        plsc.subcore_barrier()

        row_base = shard * rows_per_shard

        def make_chunk_body(acc_list):
            def chunk_body(ci, carry):
                r0 = pl.multiple_of(row_base + ci * chunk_rows, 8)
                cp_s = pltpu.async_copy(
                    src_hbm.at[pl.ds(r0, chunk_rows)], sbuf, sem_e)
                cp_d = pltpu.async_copy(
                    dst_hbm.at[pl.ds(r0, chunk_rows)], dbuf, sem_e)
                cp_w = pltpu.async_copy(
                    ew_hbm.at[pl.ds(r0, chunk_rows)], wbuf, sem_e)
                cp_s.wait()
                cp_d.wait()
                cp_w.wait()
                for j in range(chunk_rows):
                    for l in range(8):
                        s16 = sbuf[j, pl.ds(l * LANES, LANES)]
                        w16 = wbuf[j, pl.ds(l * LANES, LANES)]
                        for cc in range(cg):
                            cbase = jnp.full((LANES,), cc * N, jnp.int32)
                            v = plsc.load_gather(zv, [s16 + cbase])
                            msg[cc, j, 0, pl.ds(l * LANES, LANES)] = v * w16
                descs = []
                for cc in range(cg):
                    for j in range(chunk_rows):
                        descs.append(pltpu.async_copy(
                            msg.at[cc, j, 0],
                            acc_list[cc].at[dbuf.at[j, 0]],
                            sem_sc, add=True))
                for dsc in descs:
                    dsc.wait()
                return carry
            return chunk_body

        if kc == 1:
            lax.fori_loop(0, nchunk, make_chunk_body([accs[0]]), 0)
        else:
            for g in range(ngrp):
                @pl.when(grp == g)
                def _(g=g):
                    lax.fori_loop(
                        0, nchunk,
                        make_chunk_body([accs[g * cg + i] for i in range(cg)]),
                        0)

        plsc.subcore_barrier()

        if kc == 1:
            @pl.when(s == 0)
            def _():
                off = pl.multiple_of(c * N, 8)
                pltpu.sync_copy(accs[0], bounce)
                pltpu.sync_copy(bounce, out_hbm.at[pl.ds(off, N)])
        else:
            for r in range(acc_rows):
                @pl.when(s == r)
                def _(r=r):
                    off = pl.multiple_of((c * acc_rows + r) * N, 8)
                    pltpu.sync_copy(accs[r], bounce)
                    pltpu.sync_copy(bounce, out_hbm.at[pl.ds(off, N)])

    def call(y, src2d, dst2d, ew2d, zero_flat):
        out = sc_agg(y.reshape(-1), src2d, dst2d, ew2d, zero_flat)
        return out.reshape(out_rows, N)

    return call


@functools.lru_cache(maxsize=None)
def _sc_agg(kc):
    if kc == 1:
        return _make_sc_agg(kc=1, cg=1, ngrp=1, nshard=16, chunk_rows=8)
    return _make_sc_agg(kc=kc, cg=5, ngrp=2, nshard=8, chunk_rows=8)


def kernel(x, edge_index, edge_attr, num_graphs, Wr1, br1, Wo1, Wr2, br2, Wo2,
           Wr3, br3, Wo3, bn1_w, bn1_b, bn2_w, bn2_b, fc1_W, fc1_b,
           fc2_W, fc2_b):
    src = edge_index[0].astype(jnp.int32)
    dst = edge_index[1].astype(jnp.int32)
    ew = edge_attr[:, 2]
    pad = EROWS * 128 - E
    src2d = jnp.concatenate([src, jnp.zeros((pad,), jnp.int32)]).reshape(
        EROWS, 128)
    dst2d = jnp.concatenate([dst, jnp.zeros((pad,), jnp.int32)]).reshape(
        EROWS, 1, 128)
    ew2d = jnp.concatenate([ew, jnp.zeros((pad,), jnp.float32)]).reshape(
        EROWS, 128)
    zz = jnp.zeros((N,), jnp.float32)

    y1, obr1 = _tc_in(x, Wr1, Wo1, br1)
    agg1 = _sc_agg(H)(y1, src2d, dst2d, ew2d, zz)
    y2, obr2 = _tc_mid(agg1, obr1, bn1_w, bn1_b, Wr2, Wo2, br2)
    agg2 = _sc_agg(H)(y2, src2d, dst2d, ew2d, zz)
    y3, obr3 = _tc_mid(agg2, obr2, bn2_w, bn2_b, Wr3, Wo3, br3)
    agg3 = _sc_agg(1)(y3, src2d, dst2d, ew2d, zz)
    out, h = _tc_out(agg3, obr3, fc1_W, fc1_b, fc2_W, fc2_b)
    return (out, h)
